# baseline (speedup 1.0000x reference)
import functools

import jax
import jax.numpy as jnp
from jax import lax
from jax.experimental import pallas as pl
from jax.experimental.pallas import tpu as pltpu
from jax.experimental.pallas import tpu_sc as plsc

VOCAB = 1000
D_MODEL = 32
BATCH = 1024
SEQ = 50
SEQ_A = 16
SEQ_B = SEQ - SEQ_A
SEQ_C = 0
LANES = 16
N_VEC = BATCH // LANES

NC, NS = 2, 16
NW = NC * NS


@functools.cache
def _make_lookup_kernel(l0, n_l, split):
    mesh = plsc.VectorSubcoreMesh(core_axis_name="c", subcore_axis_name="s",
                                  num_cores=NC, num_subcores=NS)
    ncol = BATCH // split
    n_job = n_l * split

    @functools.partial(
        pl.kernel,
        out_type=jax.ShapeDtypeStruct((n_l, D_MODEL, BATCH), jnp.float32),
        mesh=mesh,
        compiler_params=pltpu.CompilerParams(needs_layout_passes=False),
        scratch_types=[
            pltpu.VMEM((D_MODEL, BATCH), jnp.float32),
            pltpu.VMEM((ncol,), jnp.int32),
            pltpu.VMEM((D_MODEL, ncol), jnp.float32),
        ],
    )
    def lookup_kernel(embt_hbm, xt_hbm, ht_hbm, tab_v, idx_v, h_v):
        wid = lax.axis_index("s") * NC + lax.axis_index("c")
        pltpu.sync_copy(embt_hbm, tab_v)

        def _one_job(job):
            l = job // split
            c0 = (job % split) * ncol
            pltpu.sync_copy(xt_hbm.at[l0 + l, pl.ds(c0, ncol)], idx_v)

            def _col_group(g, carry):
                cols = idx_v[pl.ds(g * LANES, LANES)]
                for d in range(D_MODEL):
                    rows = jnp.full((LANES,), d, dtype=jnp.int32)
                    h_v[d, pl.ds(g * LANES, LANES)] = plsc.load_gather(
                        tab_v, [rows, cols])
                return carry

            lax.fori_loop(0, ncol // LANES, _col_group, 0)
            pltpu.sync_copy(h_v, ht_hbm.at[l].at[:, pl.ds(c0, ncol)])

        for j0 in range(0, n_job, NW):
            if n_job - j0 >= NW:
                _one_job(j0 + wid)
            else:
                @pl.when(wid < n_job - j0)
                def _():
                    _one_job(j0 + wid)

    return lookup_kernel


def _proj_init_body(w_ref, b_ref, h_ref, o_ref):
    o_ref[0] = lax.dot_general(
        w_ref[...], h_ref[0], (((1,), (0,)), ((), ())),
        preferred_element_type=jnp.float32) + b_ref[...]


def _proj_update_body(w_ref, b_ref, h_ref, _, o_ref):
    o_ref[0] = lax.dot_general(
        w_ref[...], h_ref[0], (((1,), (0,)), ((), ())),
        preferred_element_type=jnp.float32) + b_ref[...]


def _project(W, b2, h_piece, l0, n_l, out_in=None):
    if out_in is None:
        return pl.pallas_call(
            _proj_init_body,
            grid=(n_l,),
            in_specs=[
                pl.BlockSpec((VOCAB, D_MODEL), lambda l: (0, 0)),
                pl.BlockSpec((VOCAB, 1), lambda l: (0, 0)),
                pl.BlockSpec((1, D_MODEL, BATCH), lambda l: (l, 0, 0)),
            ],
            out_specs=pl.BlockSpec((1, VOCAB, BATCH),
                                   lambda l: (l0 + l, 0, 0)),
            out_shape=jax.ShapeDtypeStruct((SEQ, VOCAB, BATCH), jnp.float32),
        )(W, b2, h_piece)
    return pl.pallas_call(
        _proj_update_body,
        grid=(n_l,),
        in_specs=[
            pl.BlockSpec((VOCAB, D_MODEL), lambda l: (0, 0)),
            pl.BlockSpec((VOCAB, 1), lambda l: (0, 0)),
            pl.BlockSpec((1, D_MODEL, BATCH), lambda l: (l, 0, 0)),
            pl.BlockSpec(memory_space=pl.ANY),
        ],
        out_specs=pl.BlockSpec((1, VOCAB, BATCH), lambda l: (l0 + l, 0, 0)),
        out_shape=jax.ShapeDtypeStruct((SEQ, VOCAB, BATCH), jnp.float32),
        input_output_aliases={3: 0},
    )(W, b2, h_piece, out_in)


def kernel(x, emb, W, b):
    x_t = x.T
    emb_t = jnp.pad(emb.T, ((0, 0), (0, BATCH - VOCAB)))
    h_a = _make_lookup_kernel(0, SEQ_A, 2)(emb_t, x_t)
    h_b = _make_lookup_kernel(SEQ_A, SEQ_B, 1)(emb_t, x_t)
    b2 = b.reshape(VOCAB, 1)
    out_t = _project(W, b2, h_a, 0, SEQ_A)
    out_t = _project(W, b2, h_b, SEQ_A, SEQ_B, out_t)
    return jnp.transpose(out_t, (2, 0, 1))


# speedup vs baseline: 1.0250x; 1.0250x over previous
import functools

import jax
import jax.numpy as jnp
from jax import lax
from jax.experimental import pallas as pl
from jax.experimental.pallas import tpu as pltpu
from jax.experimental.pallas import tpu_sc as plsc

VOCAB = 1000
D_MODEL = 32
BATCH = 1024
SEQ = 50
SEQ_A = 32
SEQ_B = SEQ - SEQ_A
LANES = 16
N_VEC = BATCH // LANES

NC, NS = 2, 16
NW = NC * NS


@functools.cache
def _make_lookup_kernel(l0, n_l, split):
    mesh = plsc.VectorSubcoreMesh(core_axis_name="c", subcore_axis_name="s",
                                  num_cores=NC, num_subcores=NS)
    ncol = BATCH // split
    n_job = n_l * split

    @functools.partial(
        pl.kernel,
        out_type=jax.ShapeDtypeStruct((n_l, D_MODEL, BATCH), jnp.float32),
        mesh=mesh,
        compiler_params=pltpu.CompilerParams(needs_layout_passes=False),
        scratch_types=[
            pltpu.VMEM((D_MODEL, BATCH), jnp.float32),
            pltpu.VMEM((ncol,), jnp.int32),
            pltpu.VMEM((D_MODEL, ncol), jnp.float32),
        ],
    )
    def lookup_kernel(embt_hbm, xt_hbm, ht_hbm, tab_v, idx_v, h_v):
        wid = lax.axis_index("s") * NC + lax.axis_index("c")
        pltpu.sync_copy(embt_hbm, tab_v)

        def _one_job(job):
            l = job // split
            c0 = (job % split) * ncol
            pltpu.sync_copy(xt_hbm.at[l0 + l, pl.ds(c0, ncol)], idx_v)

            def _col_group(g, carry):
                cols = idx_v[pl.ds(g * LANES, LANES)]
                for d in range(D_MODEL):
                    rows = jnp.full((LANES,), d, dtype=jnp.int32)
                    h_v[d, pl.ds(g * LANES, LANES)] = plsc.load_gather(
                        tab_v, [rows, cols])
                return carry

            lax.fori_loop(0, ncol // LANES, _col_group, 0)
            pltpu.sync_copy(h_v, ht_hbm.at[l].at[:, pl.ds(c0, ncol)])

        for j0 in range(0, n_job, NW):
            if n_job - j0 >= NW:
                _one_job(j0 + wid)
            else:
                @pl.when(wid < n_job - j0)
                def _():
                    _one_job(j0 + wid)

    return lookup_kernel


def _proj_init_body(w_ref, b_ref, h_ref, o_ref):
    o_ref[0] = lax.dot_general(
        w_ref[...], h_ref[0], (((1,), (0,)), ((), ())),
        preferred_element_type=jnp.float32) + b_ref[...]


def _proj_update_body(w_ref, b_ref, h_ref, _, o_ref):
    o_ref[0] = lax.dot_general(
        w_ref[...], h_ref[0], (((1,), (0,)), ((), ())),
        preferred_element_type=jnp.float32) + b_ref[...]


def _project(W, b2, h_piece, l0, n_l, out_in=None):
    if out_in is None:
        return pl.pallas_call(
            _proj_init_body,
            grid=(n_l,),
            in_specs=[
                pl.BlockSpec((VOCAB, D_MODEL), lambda l: (0, 0)),
                pl.BlockSpec((VOCAB, 1), lambda l: (0, 0)),
                pl.BlockSpec((1, D_MODEL, BATCH), lambda l: (l, 0, 0)),
            ],
            out_specs=pl.BlockSpec((1, VOCAB, BATCH),
                                   lambda l: (l0 + l, 0, 0)),
            out_shape=jax.ShapeDtypeStruct((SEQ, VOCAB, BATCH), jnp.float32),
        )(W, b2, h_piece)
    return pl.pallas_call(
        _proj_update_body,
        grid=(n_l,),
        in_specs=[
            pl.BlockSpec((VOCAB, D_MODEL), lambda l: (0, 0)),
            pl.BlockSpec((VOCAB, 1), lambda l: (0, 0)),
            pl.BlockSpec((1, D_MODEL, BATCH), lambda l: (l, 0, 0)),
            pl.BlockSpec(memory_space=pl.ANY),
        ],
        out_specs=pl.BlockSpec((1, VOCAB, BATCH), lambda l: (l0 + l, 0, 0)),
        out_shape=jax.ShapeDtypeStruct((SEQ, VOCAB, BATCH), jnp.float32),
        input_output_aliases={3: 0},
    )(W, b2, h_piece, out_in)


def kernel(x, emb, W, b):
    x_t = x.T
    emb_t = jnp.pad(emb.T, ((0, 0), (0, BATCH - VOCAB)))
    h_a = _make_lookup_kernel(0, SEQ_A, 1)(emb_t, x_t)
    h_b = _make_lookup_kernel(SEQ_A, SEQ_B, 1)(emb_t, x_t)
    b2 = b.reshape(VOCAB, 1)
    out_t = _project(W, b2, h_a, 0, SEQ_A)
    out_t = _project(W, b2, h_b, SEQ_A, SEQ_B, out_t)
    return jnp.transpose(out_t, (2, 0, 1))
